# baseline (speedup 1.0000x reference)
import functools

import jax
import jax.numpy as jnp
from jax.experimental import pallas as pl
from jax.experimental.pallas import tpu as pltpu

N = 5000
IN_DIM = 128
OUT_DIM = 128
DIM = 320
KS = (0.9, 0.8, 0.7)
K0 = int(KS[0] * N)
K1 = int(KS[1] * K0)
K2 = int(KS[2] * K1)

BLK = 1000
G = N // BLK

_F32 = jnp.float32
_HI = jax.lax.Precision.HIGHEST


def _dot(a, b):
    return jax.lax.dot_general(a, b, (((1,), (0,)), ((), ())),
                               precision=_HI, preferred_element_type=_F32)


def _row_spec(cols):
    return pl.BlockSpec((BLK, cols), lambda i: (i, 0))


def _full_spec(a, b):
    return pl.BlockSpec((a, b), lambda i: (0, 0))


def _down0_kernel(x_ref, ws_ref, bs_ref, wd_ref, bd_ref, wp_ref, bp_ref,
                  start_ref, cur_ref, d_ref, s_ref):
    x0 = _dot(x_ref[...], ws_ref[...]) + bs_ref[...]
    start_ref[...] = x0
    d = _dot(x0, wd_ref[...]) + bd_ref[...]
    d_ref[...] = d
    s = jax.nn.sigmoid((_dot(d, wp_ref[...]) + bp_ref[...]) / 100.0)
    s_ref[...] = s
    cur_ref[...] = d * s


def _level_kernel(cur_in_ref, act_ref, wd_ref, bd_ref, wp_ref, bp_ref,
                  cur_ref, d_ref, s_ref):
    d = _dot(cur_in_ref[...], wd_ref[...]) + bd_ref[...]
    d_ref[...] = d
    s = jax.nn.sigmoid((_dot(d, wp_ref[...]) + bp_ref[...]) / 100.0)
    s_ref[...] = jnp.where(act_ref[...] > 0.5, s, jnp.float32(-1.0))
    cur_ref[...] = d * s


def _up_all_kernel(cur3_ref, m3_ref, m2_ref, m1_ref,
                   d2_ref, d1_ref, d0_ref, x0_ref,
                   wb_ref, bb_ref, wu0_ref, bu0_ref, wu1_ref, bu1_ref,
                   wu2_ref, bu2_ref, we_a_ref, we_b_ref, be_ref, out_ref):
    bot = _dot(cur3_ref[...] * m3_ref[...], wb_ref[...]) + bb_ref[...]
    cur = _dot(bot * m3_ref[...], wu0_ref[...]) + bu0_ref[...] + d2_ref[...]
    cur = _dot(cur * m2_ref[...], wu1_ref[...]) + bu1_ref[...] + d1_ref[...]
    cur = _dot(cur * m1_ref[...], wu2_ref[...]) + bu2_ref[...] + d0_ref[...]
    out_ref[...] = (_dot(cur, we_a_ref[...])
                    + _dot(x0_ref[...], we_b_ref[...]) + be_ref[...])


def _select_kernel(s_ref, m_ref, *, k):
    bits = jax.lax.bitcast_convert_type(s_ref[...], jnp.int32)
    bc = jnp.transpose(bits)

    def value_step(i, t):
        cand = t + jnp.left_shift(jnp.int32(1), 30 - i)
        cnt = jnp.sum((bc >= cand).astype(jnp.int32))
        return jnp.where(cnt >= k, cand, t)

    t = jax.lax.fori_loop(0, 31, value_step, jnp.int32(0))

    r = k - jnp.sum((bc > t).astype(jnp.int32))
    idx_c = jax.lax.broadcasted_iota(jnp.int32, (1, N), 1)
    tie_c = bc == t

    def index_step(i, j):
        cand = j + jnp.left_shift(jnp.int32(1), 12 - i)
        cnt = jnp.sum((tie_c & (idx_c < cand)).astype(jnp.int32))
        return jnp.where(cnt <= r - 1, cand, j)

    jmax = jax.lax.fori_loop(0, 13, index_step, jnp.int32(0))
    iota = jax.lax.broadcasted_iota(jnp.int32, (N, 1), 0)
    keep = (bits > t) | ((bits == t) & (iota <= jmax))
    m_ref[...] = keep.astype(_F32)


def _grid_call(body, in_arrs, in_specs, out_cols, n_out=1, interpret=False):
    shapes = tuple(jax.ShapeDtypeStruct((N, c), _F32) for c in out_cols)
    return pl.pallas_call(
        body,
        grid=(G,),
        in_specs=in_specs,
        out_specs=tuple(_row_spec(c) for c in out_cols) if n_out > 1
        else _row_spec(out_cols[0]),
        out_shape=shapes if n_out > 1 else shapes[0],
        compiler_params=pltpu.CompilerParams(
            dimension_semantics=("parallel",)),
        interpret=interpret,
    )(*in_arrs)


@functools.partial(jax.jit, static_argnames=("interpret",))
def _run(X, W_start, b_start, W_bottom, b_bottom, W_end, b_end,
         W_down_0, b_down_0, W_down_1, b_down_1, W_down_2, b_down_2,
         W_up_0, b_up_0, W_up_1, b_up_1, W_up_2, b_up_2,
         W_pool_0, b_pool_0, W_pool_1, b_pool_1, W_pool_2, b_pool_2,
         interpret=False):
    wdim = _full_spec(DIM, DIM)
    wpool = _full_spec(DIM, 1)
    b1 = _full_spec(1, DIM)
    bp1 = _full_spec(1, 1)

    def select(s, k):
        return pl.pallas_call(
            functools.partial(_select_kernel, k=k),
            out_shape=jax.ShapeDtypeStruct((N, 1), _F32),
            interpret=interpret,
        )(s)

    start, cur1, d0, s0 = _grid_call(
        _down0_kernel,
        (X, W_start, b_start.reshape(1, DIM),
         W_down_0, b_down_0.reshape(1, DIM),
         W_pool_0, b_pool_0.reshape(1, 1)),
        (_row_spec(IN_DIM), _full_spec(IN_DIM, DIM), b1, wdim, b1, wpool, bp1),
        (DIM, DIM, DIM, 1), n_out=4, interpret=interpret)
    m1 = select(s0, K0)

    cur2, d1, s1 = _grid_call(
        _level_kernel,
        (cur1, m1, W_down_1, b_down_1.reshape(1, DIM),
         W_pool_1, b_pool_1.reshape(1, 1)),
        (_row_spec(DIM), _row_spec(1), wdim, b1, wpool, bp1),
        (DIM, DIM, 1), n_out=3, interpret=interpret)
    m2 = select(s1, K1)

    cur3, d2, s2 = _grid_call(
        _level_kernel,
        (cur2, m2, W_down_2, b_down_2.reshape(1, DIM),
         W_pool_2, b_pool_2.reshape(1, 1)),
        (_row_spec(DIM), _row_spec(1), wdim, b1, wpool, bp1),
        (DIM, DIM, 1), n_out=3, interpret=interpret)
    m3 = select(s2, K2)

    out = _grid_call(
        _up_all_kernel,
        (cur3, m3, m2, m1, d2, d1, d0, start,
         W_bottom, b_bottom.reshape(1, DIM),
         W_up_0, b_up_0.reshape(1, DIM),
         W_up_1, b_up_1.reshape(1, DIM),
         W_up_2, b_up_2.reshape(1, DIM),
         W_end[:DIM], W_end[DIM:], b_end.reshape(1, OUT_DIM)),
        (_row_spec(DIM), _row_spec(1), _row_spec(1), _row_spec(1),
         _row_spec(DIM), _row_spec(DIM), _row_spec(DIM), _row_spec(DIM),
         wdim, b1, wdim, b1, wdim, b1, wdim, b1,
         _full_spec(DIM, OUT_DIM), _full_spec(DIM, OUT_DIM),
         _full_spec(1, OUT_DIM)),
        (OUT_DIM,), interpret=interpret)
    return out, start


def kernel(A, X, W_start, b_start, W_bottom, b_bottom, W_end, b_end,
           W_down_0, b_down_0, W_down_1, b_down_1, W_down_2, b_down_2,
           W_up_0, b_up_0, W_up_1, b_up_1, W_up_2, b_up_2,
           W_pool_0, b_pool_0, W_pool_1, b_pool_1, W_pool_2, b_pool_2):
    del A
    return _run(X, W_start, b_start, W_bottom, b_bottom, W_end, b_end,
                W_down_0, b_down_0, W_down_1, b_down_1, W_down_2, b_down_2,
                W_up_0, b_up_0, W_up_1, b_up_1, W_up_2, b_up_2,
                W_pool_0, b_pool_0, W_pool_1, b_pool_1, W_pool_2, b_pool_2)


# speedup vs baseline: 1.0156x; 1.0156x over previous
import functools

import jax
import jax.numpy as jnp
from jax.experimental import pallas as pl
from jax.experimental.pallas import tpu as pltpu

N = 5000
IN_DIM = 128
OUT_DIM = 128
DIM = 320
KS = (0.9, 0.8, 0.7)
K0 = int(KS[0] * N)
K1 = int(KS[1] * K0)
K2 = int(KS[2] * K1)

BLK = 1000
G = N // BLK

_F32 = jnp.float32
_HI = jax.lax.Precision.HIGHEST


def _dot(a, b):
    return jax.lax.dot_general(a, b, (((1,), (0,)), ((), ())),
                               precision=_HI, preferred_element_type=_F32)


def _row_spec(cols):
    return pl.BlockSpec((BLK, cols), lambda i: (i, 0))


def _full_spec(a, b):
    return pl.BlockSpec((a, b), lambda i: (0, 0))


def _down0_kernel(x_ref, ws_ref, bs_ref, wd_ref, bd_ref, wp_ref, bp_ref,
                  start_ref, cur_ref, d_ref, m_ref, s_scr, *, k):
    i = pl.program_id(0)
    x0 = _dot(x_ref[...], ws_ref[...]) + bs_ref[...]
    start_ref[...] = x0
    d = _dot(x0, wd_ref[...]) + bd_ref[...]
    d_ref[...] = d
    s = jax.nn.sigmoid((_dot(d, wp_ref[...]) + bp_ref[...]) / 100.0)
    s_scr[pl.ds(i * BLK, BLK), :] = s
    cur_ref[...] = d * s

    @pl.when(i == G - 1)
    def _():
        _select_body(s_scr[...], m_ref, k)


def _level_kernel(cur_in_ref, act_ref, wd_ref, bd_ref, wp_ref, bp_ref,
                  cur_ref, d_ref, m_ref, s_scr, *, k):
    i = pl.program_id(0)
    d = _dot(cur_in_ref[...], wd_ref[...]) + bd_ref[...]
    d_ref[...] = d
    s = jax.nn.sigmoid((_dot(d, wp_ref[...]) + bp_ref[...]) / 100.0)
    s_scr[pl.ds(i * BLK, BLK), :] = jnp.where(act_ref[...] > 0.5, s,
                                              jnp.float32(-1.0))
    cur_ref[...] = d * s

    @pl.when(i == G - 1)
    def _():
        _select_body(s_scr[...], m_ref, k)


def _up_all_kernel(cur3_ref, m3_ref, m2_ref, m1_ref,
                   d2_ref, d1_ref, d0_ref, x0_ref,
                   wb_ref, bb_ref, wu0_ref, bu0_ref, wu1_ref, bu1_ref,
                   wu2_ref, bu2_ref, we_a_ref, we_b_ref, be_ref, out_ref):
    bot = _dot(cur3_ref[...] * m3_ref[...], wb_ref[...]) + bb_ref[...]
    cur = _dot(bot * m3_ref[...], wu0_ref[...]) + bu0_ref[...] + d2_ref[...]
    cur = _dot(cur * m2_ref[...], wu1_ref[...]) + bu1_ref[...] + d1_ref[...]
    cur = _dot(cur * m1_ref[...], wu2_ref[...]) + bu2_ref[...] + d0_ref[...]
    out_ref[...] = (_dot(cur, we_a_ref[...])
                    + _dot(x0_ref[...], we_b_ref[...]) + be_ref[...])


def _select_body(s, m_ref, k):
    bits = jax.lax.bitcast_convert_type(s, jnp.int32)
    bc = jnp.transpose(bits)

    def value_step(i, t):
        cand = t + jnp.left_shift(jnp.int32(1), 30 - i)
        cnt = jnp.sum((bc >= cand).astype(jnp.int32))
        return jnp.where(cnt >= k, cand, t)

    t = jax.lax.fori_loop(0, 31, value_step, jnp.int32(0))

    r = k - jnp.sum((bc > t).astype(jnp.int32))
    idx_c = jax.lax.broadcasted_iota(jnp.int32, (1, N), 1)
    tie_c = bc == t

    def index_step(i, j):
        cand = j + jnp.left_shift(jnp.int32(1), 12 - i)
        cnt = jnp.sum((tie_c & (idx_c < cand)).astype(jnp.int32))
        return jnp.where(cnt <= r - 1, cand, j)

    jmax = jax.lax.fori_loop(0, 13, index_step, jnp.int32(0))
    iota = jax.lax.broadcasted_iota(jnp.int32, (N, 1), 0)
    keep = (bits > t) | ((bits == t) & (iota <= jmax))
    m_ref[...] = keep.astype(_F32)


def _grid_call(body, in_arrs, in_specs, out_cols, n_out=1, interpret=False):
    shapes = tuple(jax.ShapeDtypeStruct((N, c), _F32) for c in out_cols)
    return pl.pallas_call(
        body,
        grid=(G,),
        in_specs=in_specs,
        out_specs=tuple(_row_spec(c) for c in out_cols) if n_out > 1
        else _row_spec(out_cols[0]),
        out_shape=shapes if n_out > 1 else shapes[0],
        interpret=interpret,
    )(*in_arrs)


def _level_call(body, k, in_arrs, in_specs, interpret=False):
    n_row_outs = len(in_specs) - 4
    out_specs = (tuple(_row_spec(DIM) for _ in range(n_row_outs))
                 + (pl.BlockSpec((N, 1), lambda i: (0, 0)),))
    out_shape = (tuple(jax.ShapeDtypeStruct((N, DIM), _F32)
                       for _ in range(n_row_outs))
                 + (jax.ShapeDtypeStruct((N, 1), _F32),))
    return pl.pallas_call(
        functools.partial(body, k=k),
        grid=(G,),
        in_specs=in_specs,
        out_specs=out_specs,
        out_shape=out_shape,
        scratch_shapes=[pltpu.VMEM((N, 1), _F32)],
        interpret=interpret,
    )(*in_arrs)


@functools.partial(jax.jit, static_argnames=("interpret",))
def _run(X, W_start, b_start, W_bottom, b_bottom, W_end, b_end,
         W_down_0, b_down_0, W_down_1, b_down_1, W_down_2, b_down_2,
         W_up_0, b_up_0, W_up_1, b_up_1, W_up_2, b_up_2,
         W_pool_0, b_pool_0, W_pool_1, b_pool_1, W_pool_2, b_pool_2,
         interpret=False):
    wdim = _full_spec(DIM, DIM)
    wpool = _full_spec(DIM, 1)
    b1 = _full_spec(1, DIM)
    bp1 = _full_spec(1, 1)

    start, cur1, d0, m1 = _level_call(
        _down0_kernel, K0,
        (X, W_start, b_start.reshape(1, DIM),
         W_down_0, b_down_0.reshape(1, DIM),
         W_pool_0, b_pool_0.reshape(1, 1)),
        (_row_spec(IN_DIM), _full_spec(IN_DIM, DIM), b1, wdim, b1, wpool, bp1),
        interpret=interpret)

    cur2, d1, m2 = _level_call(
        _level_kernel, K1,
        (cur1, m1, W_down_1, b_down_1.reshape(1, DIM),
         W_pool_1, b_pool_1.reshape(1, 1)),
        (_row_spec(DIM), _row_spec(1), wdim, b1, wpool, bp1),
        interpret=interpret)

    cur3, d2, m3 = _level_call(
        _level_kernel, K2,
        (cur2, m2, W_down_2, b_down_2.reshape(1, DIM),
         W_pool_2, b_pool_2.reshape(1, 1)),
        (_row_spec(DIM), _row_spec(1), wdim, b1, wpool, bp1),
        interpret=interpret)

    out = _grid_call(
        _up_all_kernel,
        (cur3, m3, m2, m1, d2, d1, d0, start,
         W_bottom, b_bottom.reshape(1, DIM),
         W_up_0, b_up_0.reshape(1, DIM),
         W_up_1, b_up_1.reshape(1, DIM),
         W_up_2, b_up_2.reshape(1, DIM),
         W_end[:DIM], W_end[DIM:], b_end.reshape(1, OUT_DIM)),
        (_row_spec(DIM), _row_spec(1), _row_spec(1), _row_spec(1),
         _row_spec(DIM), _row_spec(DIM), _row_spec(DIM), _row_spec(DIM),
         wdim, b1, wdim, b1, wdim, b1, wdim, b1,
         _full_spec(DIM, OUT_DIM), _full_spec(DIM, OUT_DIM),
         _full_spec(1, OUT_DIM)),
        (OUT_DIM,), interpret=interpret)
    return out, start


def kernel(A, X, W_start, b_start, W_bottom, b_bottom, W_end, b_end,
           W_down_0, b_down_0, W_down_1, b_down_1, W_down_2, b_down_2,
           W_up_0, b_up_0, W_up_1, b_up_1, W_up_2, b_up_2,
           W_pool_0, b_pool_0, W_pool_1, b_pool_1, W_pool_2, b_pool_2):
    del A
    return _run(X, W_start, b_start, W_bottom, b_bottom, W_end, b_end,
                W_down_0, b_down_0, W_down_1, b_down_1, W_down_2, b_down_2,
                W_up_0, b_up_0, W_up_1, b_up_1, W_up_2, b_up_2,
                W_pool_0, b_pool_0, W_pool_1, b_pool_1, W_pool_2, b_pool_2)
